# baseline (speedup 1.0000x reference)
import jax
import jax.numpy as jnp
from jax import lax
from jax.experimental import pallas as pl
from jax.experimental.pallas import tpu as pltpu
from jax.experimental.pallas import tpu_sc as plsc

NC = 2
NS = 16
L = 16
NW = NC * NS
B = 16384
K = 32
BPW = B // NW
CHUNK = 128
NCHUNK = BPW // CHUNK


def _mf_body(user_hbm, item_hbm, ut_hbm, it_hbm, out_hbm,
             uidx_v, iidx_v, urows_v, irows_v, out_v, sem_u, sem_i):
    wid = lax.axis_index("s") * NC + lax.axis_index("c")
    base = wid * BPW

    for j in range(NCHUNK):
        pltpu.sync_copy(user_hbm.at[pl.ds(base + j * CHUNK, CHUNK)], uidx_v.at[j])
        pltpu.sync_copy(item_hbm.at[pl.ds(base + j * CHUNK, CHUNK)], iidx_v.at[j])

    copies = []
    for j in range(NCHUNK):
        copies.append(pltpu.async_copy(
            ut_hbm.at[uidx_v.at[j]], urows_v.at[pl.ds(j * CHUNK, CHUNK)], sem_u))
        copies.append(pltpu.async_copy(
            it_hbm.at[iidx_v.at[j]], irows_v.at[pl.ds(j * CHUNK, CHUNK)], sem_i))
    for c in copies:
        c.wait()

    iota = lax.iota(jnp.int32, L)

    def body(c, carry):
        rows = c * L + iota
        acc = jnp.zeros((L,), jnp.float32)
        for k in range(K):
            col = jnp.full((L,), k, jnp.int32)
            uk = plsc.load_gather(urows_v, [rows, col])
            ik = plsc.load_gather(irows_v, [rows, col])
            acc = acc + uk * ik
        out_v[pl.ds(c * L, L)] = 1.0 / (1.0 + jnp.exp(-acc))
        return carry

    lax.fori_loop(0, BPW // L, body, 0)

    pltpu.sync_copy(out_v, out_hbm.at[pl.ds(base, BPW)])


def kernel(user, item, user_table, item_table):
    mesh = plsc.VectorSubcoreMesh(core_axis_name="c", subcore_axis_name="s")
    run = pl.kernel(
        _mf_body,
        mesh=mesh,
        out_type=jax.ShapeDtypeStruct((B,), jnp.float32),
        scratch_types=[
            pltpu.VMEM((NCHUNK, CHUNK), jnp.int32),
            pltpu.VMEM((NCHUNK, CHUNK), jnp.int32),
            pltpu.VMEM((BPW, K), jnp.float32),
            pltpu.VMEM((BPW, K), jnp.float32),
            pltpu.VMEM((BPW,), jnp.float32),
            pltpu.SemaphoreType.DMA,
            pltpu.SemaphoreType.DMA,
        ],
        compiler_params=pltpu.CompilerParams(
            needs_layout_passes=False, use_tc_tiling_on_sc=False),
    )
    return run(user.astype(jnp.int32), item.astype(jnp.int32),
               user_table, item_table)


# speedup vs baseline: 3.5412x; 3.5412x over previous
import jax
import jax.numpy as jnp
from jax import lax
from jax.experimental import pallas as pl
from jax.experimental.pallas import tpu as pltpu
from jax.experimental.pallas import tpu_sc as plsc

NC = 2
NS = 16
L = 16
NW = NC * NS
B = 16384
K = 32
BPW = B // NW
W = 128
SUB = 4


def _mf_body(user_hbm, item_hbm, ut_hbm, it_hbm, out_hbm,
             uidx_v, iidx_v, uslab, islab, out_v, sem_a, sem_b):
    cid = lax.axis_index("c")
    sid = lax.axis_index("s")
    wid = sid * NC + cid
    base = wid * BPW

    pltpu.sync_copy(user_hbm.at[pl.ds(base, BPW)], uidx_v)
    pltpu.sync_copy(item_hbm.at[pl.ds(base, BPW)], iidx_v)

    iota = lax.iota(jnp.int32, L)
    sems = [sem_a, sem_b]

    def fire(uvec, ivec, sub, buf):
        sem = sems[buf]
        for i in range(SUB):
            r_u = uvec[sub * SUB + i]
            r_i = ivec[sub * SUB + i]
            cb_u = pl.multiple_of(r_u - lax.rem(r_u, W), W)
            cb_i = pl.multiple_of(r_i - lax.rem(r_i, W), W)
            pltpu.async_copy(ut_hbm.at[:, pl.ds(cb_u, W)],
                             uslab.at[buf, i], sem)
            pltpu.async_copy(it_hbm.at[:, pl.ds(cb_i, W)],
                             islab.at[buf, i], sem)

    def drain(buf):
        sem = sems[buf]
        for _ in range(2 * SUB):
            pltpu.make_async_copy(
                ut_hbm.at[:, pl.ds(0, W)], uslab.at[buf, 0], sem).wait()

    def compute(uvec, ivec, sub, buf, acc):
        for i in range(SUB):
            r_u = uvec[sub * SUB + i]
            r_i = ivec[sub * SUB + i]
            j_u = lax.rem(r_u, W)
            j_i = lax.rem(r_i, W)
            cu = jnp.full((L,), j_u, jnp.int32)
            ci = jnp.full((L,), j_i, jnp.int32)
            u_lo = plsc.load_gather(uslab, [jnp.full((L,), buf, jnp.int32),
                                            jnp.full((L,), i, jnp.int32),
                                            iota, cu])
            u_hi = plsc.load_gather(uslab, [jnp.full((L,), buf, jnp.int32),
                                            jnp.full((L,), i, jnp.int32),
                                            iota + L, cu])
            i_lo = plsc.load_gather(islab, [jnp.full((L,), buf, jnp.int32),
                                            jnp.full((L,), i, jnp.int32),
                                            iota, ci])
            i_hi = plsc.load_gather(islab, [jnp.full((L,), buf, jnp.int32),
                                            jnp.full((L,), i, jnp.int32),
                                            iota + L, ci])
            prod = u_lo * i_lo + u_hi * i_hi
            s = lax.reduce_sum_p.bind(prod, axes=(0,))
            lane = sub * SUB + i
            acc = jnp.where(iota == lane, s, acc)
        return acc

    nsub = L // SUB

    def block_body(g, carry):
        uvec = uidx_v[pl.ds(g * L, L)]
        ivec = iidx_v[pl.ds(g * L, L)]
        acc = jnp.zeros((L,), jnp.float32)
        fire(uvec, ivec, 0, 0)
        for sub in range(nsub):
            if sub + 1 < nsub:
                fire(uvec, ivec, sub + 1, (sub + 1) % 2)
            drain(sub % 2)
            acc = compute(uvec, ivec, sub, sub % 2, acc)
        out_v[pl.ds(g * L, L)] = 1.0 / (1.0 + jnp.exp(-acc))
        return carry

    lax.fori_loop(0, BPW // L, block_body, 0)

    pltpu.sync_copy(out_v, out_hbm.at[pl.ds(base, BPW)])


def kernel(user, item, user_table, item_table):
    mesh = plsc.VectorSubcoreMesh(core_axis_name="c", subcore_axis_name="s")
    run = pl.kernel(
        _mf_body,
        mesh=mesh,
        out_type=jax.ShapeDtypeStruct((B,), jnp.float32),
        scratch_types=[
            pltpu.VMEM((BPW,), jnp.int32),
            pltpu.VMEM((BPW,), jnp.int32),
            pltpu.VMEM((2, SUB, K, W), jnp.float32),
            pltpu.VMEM((2, SUB, K, W), jnp.float32),
            pltpu.VMEM((BPW,), jnp.float32),
            pltpu.SemaphoreType.DMA,
            pltpu.SemaphoreType.DMA,
        ],
        compiler_params=pltpu.CompilerParams(needs_layout_passes=False),
    )
    return run(user.astype(jnp.int32), item.astype(jnp.int32),
               user_table.T, item_table.T)


# speedup vs baseline: 4.1768x; 1.1795x over previous
import jax
import jax.numpy as jnp
from jax import lax
from jax.experimental import pallas as pl
from jax.experimental.pallas import tpu as pltpu
from jax.experimental.pallas import tpu_sc as plsc

NC = 2
NS = 16
L = 16
NW = NC * NS
B = 16384
K = 32
BPW = B // NW
W = 128
NSLOT = 17
CHUNK = 128
NCHUNK = BPW // CHUNK


def _extract_body(su_hbm, si_hbm, nfu_hbm, nfi_hbm, wdu_hbm, wdi_hbm,
                  ut_hbm, it_hbm, uemb_hbm, iemb_hbm,
                  idx_v, nf_v, wd_v, slabs, ebuf, sem):
    cid = lax.axis_index("c")
    sid = lax.axis_index("s")
    wid = sid * NC + cid
    base = wid * BPW
    iota = lax.iota(jnp.int32, L)

    def one_pass(sv_hbm, nf_hbm, wdx_hbm, tab, out_hbm):
        pltpu.sync_copy(sv_hbm.at[pl.ds(base, BPW)], idx_v)
        pltpu.sync_copy(nf_hbm.at[pl.ds(base, BPW)], nf_v)
        pltpu.sync_copy(wdx_hbm.at[pl.ds(base, BPW)], wd_v)

        def block(g, carry):
            svv = idx_v[pl.ds(g * L, L)]
            nfv = nf_v[pl.ds(g * L, L)]
            wdv = wd_v[pl.ds(g * L, L)]
            for i in range(L):
                r = svv[i]
                slot = lax.rem(wdv[i], NSLOT)
                cb = pl.multiple_of(r - lax.rem(r, W), W)

                @pl.when(nfv[i] == 1)
                def _fire():
                    pltpu.async_copy(tab.at[:, pl.ds(cb, W)],
                                     slabs.at[slot], sem)
            for i in range(L):
                @pl.when(nfv[i] == 1)
                def _drain():
                    pltpu.make_async_copy(tab.at[:, pl.ds(0, W)],
                                          slabs.at[0], sem).wait()
            for i in range(L):
                r = svv[i]
                slot = lax.rem(wdv[i], NSLOT)
                j = lax.rem(r, W)
                sl = jnp.full((L,), slot, jnp.int32)
                jj = jnp.full((L,), j, jnp.int32)
                lo = plsc.load_gather(slabs, [sl, iota, jj])
                hi = plsc.load_gather(slabs, [sl, iota + L, jj])
                q = g * L + i
                ebuf[pl.ds(q * K, L)] = lo
                ebuf[pl.ds(q * K + L, L)] = hi
            return carry

        lax.fori_loop(0, BPW // L, block, 0)
        pltpu.sync_copy(ebuf, out_hbm.at[pl.ds(base * K, BPW * K)])

    one_pass(su_hbm, nfu_hbm, wdu_hbm, ut_hbm, uemb_hbm)
    one_pass(si_hbm, nfi_hbm, wdi_hbm, it_hbm, iemb_hbm)


def _pair_body(upos_hbm, ipos_hbm, ue_hbm, ie_hbm, out_hbm,
               uidx_v, iidx_v, urows_v, irows_v, out_v, sem_u, sem_i):
    cid = lax.axis_index("c")
    sid = lax.axis_index("s")
    wid = sid * NC + cid
    base = wid * BPW

    for j in range(NCHUNK):
        pltpu.sync_copy(upos_hbm.at[pl.ds(base + j * CHUNK, CHUNK)],
                        uidx_v.at[j])
        pltpu.sync_copy(ipos_hbm.at[pl.ds(base + j * CHUNK, CHUNK)],
                        iidx_v.at[j])
    copies = []
    for j in range(NCHUNK):
        copies.append(pltpu.async_copy(
            ue_hbm.at[uidx_v.at[j]], urows_v.at[pl.ds(j * CHUNK, CHUNK)],
            sem_u))
        copies.append(pltpu.async_copy(
            ie_hbm.at[iidx_v.at[j]], irows_v.at[pl.ds(j * CHUNK, CHUNK)],
            sem_i))
    for c in copies:
        c.wait()

    iota = lax.iota(jnp.int32, L)

    def body(c, carry):
        rows = c * L + iota
        acc = jnp.zeros((L,), jnp.float32)
        for k in range(K):
            col = jnp.full((L,), k, jnp.int32)
            uk = plsc.load_gather(urows_v, [rows, col])
            ik = plsc.load_gather(irows_v, [rows, col])
            acc = acc + uk * ik
        out_v[pl.ds(c * L, L)] = 1.0 / (1.0 + jnp.exp(-acc))
        return carry

    lax.fori_loop(0, BPW // L, body, 0)
    pltpu.sync_copy(out_v, out_hbm.at[pl.ds(base, BPW)])


def _run_flags(sorted_idx):
    bkt = sorted_idx >> 7
    head = jnp.concatenate([
        jnp.ones((1,), jnp.int32),
        (bkt[1:] != bkt[:-1]).astype(jnp.int32)])
    pos = lax.iota(jnp.int32, B)
    head = jnp.where(lax.rem(pos, BPW) == 0, 1, head)
    wd = jnp.cumsum(head) - 1
    return head, wd.astype(jnp.int32)


def kernel(user, item, user_table, item_table):
    user = user.astype(jnp.int32)
    item = item.astype(jnp.int32)
    pu = jnp.argsort(user)
    pi = jnp.argsort(item)
    su = user[pu]
    si = item[pi]
    nfu, wdu = _run_flags(su)
    nfi, wdi = _run_flags(si)
    pos = lax.iota(jnp.int32, B)
    upos = jnp.zeros((B,), jnp.int32).at[pu].set(pos)
    ipos = jnp.zeros((B,), jnp.int32).at[pi].set(pos)

    mesh = plsc.VectorSubcoreMesh(core_axis_name="c", subcore_axis_name="s")
    extract = pl.kernel(
        _extract_body,
        mesh=mesh,
        out_type=[jax.ShapeDtypeStruct((B * K,), jnp.float32),
                  jax.ShapeDtypeStruct((B * K,), jnp.float32)],
        scratch_types=[
            pltpu.VMEM((BPW,), jnp.int32),
            pltpu.VMEM((BPW,), jnp.int32),
            pltpu.VMEM((BPW,), jnp.int32),
            pltpu.VMEM((NSLOT, K, W), jnp.float32),
            pltpu.VMEM((BPW * K,), jnp.float32),
            pltpu.SemaphoreType.DMA,
        ],
        compiler_params=pltpu.CompilerParams(needs_layout_passes=False),
    )
    uemb, iemb = extract(su, si, nfu, nfi, wdu, wdi,
                         user_table.T, item_table.T)

    pair = pl.kernel(
        _pair_body,
        mesh=mesh,
        out_type=jax.ShapeDtypeStruct((B,), jnp.float32),
        scratch_types=[
            pltpu.VMEM((NCHUNK, CHUNK), jnp.int32),
            pltpu.VMEM((NCHUNK, CHUNK), jnp.int32),
            pltpu.VMEM((BPW, K), jnp.float32),
            pltpu.VMEM((BPW, K), jnp.float32),
            pltpu.VMEM((BPW,), jnp.float32),
            pltpu.SemaphoreType.DMA,
            pltpu.SemaphoreType.DMA,
        ],
        compiler_params=pltpu.CompilerParams(
            needs_layout_passes=False, use_tc_tiling_on_sc=False),
    )
    return pair(upos, ipos, uemb.reshape(B, K), iemb.reshape(B, K))
